# baseline (speedup 1.0000x reference)
import functools

import jax
import jax.numpy as jnp
from jax import lax
from jax.experimental import pallas as pl
from jax.experimental.pallas import tpu as pltpu
from jax.experimental.pallas import tpu_sc as plsc

NUM_UNITS = 768
NUM_REL = 129


def _zero_row0_body(table_ref, out_ref):
    rows = lax.broadcasted_iota(jnp.int32, table_ref.shape, 0)
    out_ref[...] = jnp.where(rows == 0, jnp.float32(0.0), table_ref[...])


def _zero_row0(table):
    return pl.pallas_call(
        _zero_row0_body,
        out_shape=jax.ShapeDtypeStruct(table.shape, table.dtype),
    )(table)


def _onehot_body(idx_ref, table_ref, out_ref):
    idx = idx_ref[...]
    classes = lax.broadcasted_iota(jnp.int32, (1, NUM_REL), 1)
    oh = jnp.where((idx == classes) & (idx >= 1), jnp.float32(1.0),
                   jnp.float32(0.0))
    out_ref[...] = jnp.dot(oh, table_ref[...],
                           preferred_element_type=jnp.float32)


@functools.lru_cache(maxsize=None)
def _make_tc_gather(B, D):
    RB = 4096
    grid = (B // RB,)
    return pl.pallas_call(
        _onehot_body,
        grid=grid,
        in_specs=[
            pl.BlockSpec((RB, 1), lambda i: (i, 0)),
            pl.BlockSpec((NUM_REL, D), lambda i: (0, 0)),
        ],
        out_specs=pl.BlockSpec((RB, D), lambda i: (i, 0)),
        out_shape=jax.ShapeDtypeStruct((B, D), jnp.float32),
    )


@functools.lru_cache(maxsize=None)
def _make_sc_gather(B, D):
    info = plsc.get_sparse_core_info()
    NC, NS = info.num_cores, info.num_subcores
    NW = NC * NS
    b_per_w = B // NW
    C = 32
    NBUF = 4
    AHEAD = 2
    nch = b_per_w // C
    assert b_per_w % C == 0 and nch % NBUF == 0

    mesh = plsc.VectorSubcoreMesh(core_axis_name="c", subcore_axis_name="s")

    @functools.partial(
        pl.kernel,
        mesh=mesh,
        out_type=jax.ShapeDtypeStruct((B, D), jnp.float32),
        scratch_types=(
            [pltpu.VMEM((b_per_w,), jnp.int32)]
            + [pltpu.VMEM((C, D), jnp.float32)] * NBUF
            + [pltpu.SemaphoreType.DMA] * (2 * NBUF)
        ),
    )
    def gather_kernel(table_hbm, idx_hbm, out_hbm, idx_v, *bufs_and_sems):
        rows = bufs_and_sems[:NBUF]
        gsem = bufs_and_sems[NBUF:2 * NBUF]
        wsem = bufs_and_sems[2 * NBUF:]
        wid = lax.axis_index("s") * NC + lax.axis_index("c")
        base = wid * b_per_w
        pltpu.sync_copy(idx_hbm.at[pl.ds(base, b_per_w)], idx_v)

        def g_copy(c, b):
            return pltpu.make_async_copy(
                table_hbm.at[idx_v.at[pl.ds(c * C, C)]], rows[b], gsem[b])

        def w_copy(c, b):
            return pltpu.make_async_copy(
                rows[b], out_hbm.at[pl.ds(base + c * C, C)], wsem[b])

        for k in range(AHEAD):
            g_copy(k, k).start()

        def loop_body(g):
            for b in range(NBUF):
                c = g + b
                g_copy(c, b).wait()
                w_copy(c, b).start()

                @pl.when(c >= AHEAD)
                def _():
                    w_copy(c - AHEAD, (b - AHEAD) % NBUF).wait()

                @pl.when(c + AHEAD < nch)
                def _():
                    g_copy(c + AHEAD, (b + AHEAD) % NBUF).start()

        pl.loop(0, nch, step=NBUF)(loop_body)
        for k in range(AHEAD):
            c = nch - AHEAD + k
            w_copy(c, c % NBUF).wait()

    return gather_kernel


SC_FRAC_NUM = 1
SC_FRAC_DEN = 2


def kernel(relation_matrix, embeddings_table):
    bsz, seq, seq2 = relation_matrix.shape
    num_units = embeddings_table.shape[1]
    B = bsz * seq * seq2
    Bsc = (B * SC_FRAC_NUM // SC_FRAC_DEN) // 8192 * 8192
    idx = relation_matrix.reshape(-1)
    table = _zero_row0(embeddings_table)
    out_sc = _make_sc_gather(Bsc, num_units)(table, idx[:Bsc])
    out_tc = _make_tc_gather(B - Bsc, num_units)(
        idx[Bsc:].reshape(-1, 1), table)
    out = jnp.concatenate([out_sc, out_tc], axis=0)
    return out.reshape(bsz, seq, seq2, num_units)


# speedup vs baseline: 1.6704x; 1.6704x over previous
import functools

import jax
import jax.numpy as jnp
from jax import lax
from jax.experimental import pallas as pl
from jax.experimental.pallas import tpu as pltpu
from jax.experimental.pallas import tpu_sc as plsc

NUM_UNITS = 768
NUM_REL = 129


def _zero_row0_body(table_ref, out_ref):
    rows = lax.broadcasted_iota(jnp.int32, table_ref.shape, 0)
    out_ref[...] = jnp.where(rows == 0, jnp.float32(0.0), table_ref[...])


def _zero_row0(table):
    return pl.pallas_call(
        _zero_row0_body,
        out_shape=jax.ShapeDtypeStruct(table.shape, table.dtype),
    )(table)


def _onehot_body(idx_ref, table_ref, out_ref):
    idx = idx_ref[...]
    classes = lax.broadcasted_iota(jnp.int32, (1, NUM_REL), 1)
    oh = jnp.where((idx == classes) & (idx >= 1), jnp.float32(1.0),
                   jnp.float32(0.0))
    out_ref[...] = jnp.dot(oh, table_ref[...],
                           preferred_element_type=jnp.float32)


def _onehot_fill_body(idx_ref, table_ref, _prev_ref, out_ref):
    _onehot_body(idx_ref, table_ref, out_ref)


@functools.lru_cache(maxsize=None)
def _make_tc_fill(B, Bsc, D):
    RB = 4096
    nblk0 = Bsc // RB
    grid = ((B - Bsc) // RB,)
    return pl.pallas_call(
        _onehot_fill_body,
        grid=grid,
        in_specs=[
            pl.BlockSpec((RB, 1), lambda i: (i + nblk0, 0)),
            pl.BlockSpec((NUM_REL, D), lambda i: (0, 0)),
            pl.BlockSpec((8, 128), lambda i: (0, 0)),
        ],
        out_specs=pl.BlockSpec((RB, D), lambda i: (i + nblk0, 0)),
        out_shape=jax.ShapeDtypeStruct((B, D), jnp.float32),
        input_output_aliases={2: 0},
    )


@functools.lru_cache(maxsize=None)
def _make_sc_gather(B, Bsc, D):
    info = plsc.get_sparse_core_info()
    NC, NS = info.num_cores, info.num_subcores
    NW = NC * NS
    b_per_w = Bsc // NW
    C = 32
    NBUF = 4
    AHEAD = 2
    nch = b_per_w // C
    assert b_per_w % C == 0 and nch % NBUF == 0
    assert Bsc % NW == 0

    mesh = plsc.VectorSubcoreMesh(core_axis_name="c", subcore_axis_name="s")

    @functools.partial(
        pl.kernel,
        mesh=mesh,
        out_type=jax.ShapeDtypeStruct((B, D), jnp.float32),
        scratch_types=(
            [pltpu.VMEM((b_per_w,), jnp.int32)]
            + [pltpu.VMEM((C, D), jnp.float32)] * NBUF
            + [pltpu.SemaphoreType.DMA] * (2 * NBUF)
        ),
    )
    def gather_kernel(table_hbm, idx_hbm, out_hbm, idx_v, *bufs_and_sems):
        rows = bufs_and_sems[:NBUF]
        gsem = bufs_and_sems[NBUF:2 * NBUF]
        wsem = bufs_and_sems[2 * NBUF:]
        wid = lax.axis_index("s") * NC + lax.axis_index("c")
        base = wid * b_per_w
        pltpu.sync_copy(idx_hbm.at[pl.ds(base, b_per_w)], idx_v)

        def g_copy(c, b):
            return pltpu.make_async_copy(
                table_hbm.at[idx_v.at[pl.ds(c * C, C)]], rows[b], gsem[b])

        def w_copy(c, b):
            return pltpu.make_async_copy(
                rows[b], out_hbm.at[pl.ds(base + c * C, C)], wsem[b])

        for k in range(AHEAD):
            g_copy(k, k).start()

        def loop_body(g):
            for b in range(NBUF):
                c = g + b
                g_copy(c, b).wait()
                w_copy(c, b).start()

                @pl.when(c >= AHEAD)
                def _():
                    w_copy(c - AHEAD, (b - AHEAD) % NBUF).wait()

                @pl.when(c + AHEAD < nch)
                def _():
                    g_copy(c + AHEAD, (b + AHEAD) % NBUF).start()

        pl.loop(0, nch, step=NBUF)(loop_body)
        for k in range(AHEAD):
            c = nch - AHEAD + k
            w_copy(c, c % NBUF).wait()

    return gather_kernel


SC_FRAC_NUM = 1
SC_FRAC_DEN = 2


def kernel(relation_matrix, embeddings_table):
    bsz, seq, seq2 = relation_matrix.shape
    num_units = embeddings_table.shape[1]
    B = bsz * seq * seq2
    Bsc = (B * SC_FRAC_NUM // SC_FRAC_DEN) // 8192 * 8192
    idx = relation_matrix.reshape(-1)
    table = _zero_row0(embeddings_table)
    out_sc = _make_sc_gather(B, Bsc, num_units)(table, idx[:Bsc])
    out = _make_tc_fill(B, Bsc, num_units)(idx.reshape(-1, 1), table, out_sc)
    return out.reshape(bsz, seq, seq2, num_units)


# speedup vs baseline: 2.2566x; 1.3510x over previous
import functools

import jax
import jax.numpy as jnp
from jax import lax
from jax.experimental import pallas as pl
from jax.experimental.pallas import tpu as pltpu
from jax.experimental.pallas import tpu_sc as plsc

NUM_UNITS = 768
NUM_REL = 129


def _zero_row0_body(table_ref, out_ref):
    rows = lax.broadcasted_iota(jnp.int32, table_ref.shape, 0)
    out_ref[...] = jnp.where(rows == 0, jnp.float32(0.0), table_ref[...])


def _zero_row0(table):
    return pl.pallas_call(
        _zero_row0_body,
        out_shape=jax.ShapeDtypeStruct(table.shape, table.dtype),
    )(table)


def _onehot_body(idx_ref, table_ref, out_ref):
    idx = idx_ref[...]
    classes = lax.broadcasted_iota(jnp.int32, (1, NUM_REL), 1)
    oh = jnp.where((idx == classes) & (idx >= 1), jnp.float32(1.0),
                   jnp.float32(0.0))
    out_ref[...] = jnp.dot(oh, table_ref[...],
                           preferred_element_type=jnp.float32)


def _onehot_fill_body(idx_ref, table_ref, _prev_ref, out_ref):
    _onehot_body(idx_ref, table_ref, out_ref)


@functools.lru_cache(maxsize=None)
def _make_tc_fill(B, Bsc, D):
    RB = 4096
    nblk0 = Bsc // RB
    grid = ((B - Bsc) // RB,)
    return pl.pallas_call(
        _onehot_fill_body,
        grid=grid,
        in_specs=[
            pl.BlockSpec((RB, 1), lambda i: (i + nblk0, 0)),
            pl.BlockSpec((NUM_REL, D), lambda i: (0, 0)),
            pl.BlockSpec((8, 128), lambda i: (0, 0)),
        ],
        out_specs=pl.BlockSpec((RB, D), lambda i: (i + nblk0, 0)),
        out_shape=jax.ShapeDtypeStruct((B, D), jnp.float32),
        input_output_aliases={2: 0},
    )


@functools.lru_cache(maxsize=None)
def _make_sc_gather(B, Bsc, D):
    info = plsc.get_sparse_core_info()
    NC, NS = info.num_cores, info.num_subcores
    NW = NC * NS
    b_per_w = Bsc // NW
    C = 32
    NBUF = 4
    AHEAD = 2
    nch = b_per_w // C
    assert b_per_w % C == 0 and nch % NBUF == 0
    assert Bsc % NW == 0

    mesh = plsc.VectorSubcoreMesh(core_axis_name="c", subcore_axis_name="s")

    @functools.partial(
        pl.kernel,
        mesh=mesh,
        out_type=jax.ShapeDtypeStruct((B, D), jnp.float32),
        scratch_types=(
            [pltpu.VMEM((b_per_w,), jnp.int32)]
            + [pltpu.VMEM((C, D), jnp.float32)] * NBUF
            + [pltpu.SemaphoreType.DMA] * (2 * NBUF)
        ),
    )
    def gather_kernel(table_hbm, idx_hbm, out_hbm, idx_v, *bufs_and_sems):
        rows = bufs_and_sems[:NBUF]
        gsem = bufs_and_sems[NBUF:2 * NBUF]
        wsem = bufs_and_sems[2 * NBUF:]
        wid = lax.axis_index("s") * NC + lax.axis_index("c")
        base = wid * b_per_w
        pltpu.sync_copy(idx_hbm.at[pl.ds(base, b_per_w)], idx_v)

        def g_copy(c, b):
            return pltpu.make_async_copy(
                table_hbm.at[idx_v.at[pl.ds(c * C, C)]], rows[b], gsem[b])

        def w_copy(c, b):
            return pltpu.make_async_copy(
                rows[b], out_hbm.at[pl.ds(base + c * C, C)], wsem[b])

        for k in range(AHEAD):
            g_copy(k, k).start()

        def loop_body(g):
            for b in range(NBUF):
                c = g + b
                g_copy(c, b).wait()
                w_copy(c, b).start()

                @pl.when(c >= AHEAD)
                def _():
                    w_copy(c - AHEAD, (b - AHEAD) % NBUF).wait()

                @pl.when(c + AHEAD < nch)
                def _():
                    g_copy(c + AHEAD, (b + AHEAD) % NBUF).start()

        pl.loop(0, nch, step=NBUF)(loop_body)
        for k in range(AHEAD):
            c = nch - AHEAD + k
            w_copy(c, c % NBUF).wait()

    return gather_kernel


SC_FRAC_NUM = 1
SC_FRAC_DEN = 4


def kernel(relation_matrix, embeddings_table):
    bsz, seq, seq2 = relation_matrix.shape
    num_units = embeddings_table.shape[1]
    B = bsz * seq * seq2
    Bsc = (B * SC_FRAC_NUM // SC_FRAC_DEN) // 8192 * 8192
    idx = relation_matrix.reshape(-1)
    table = _zero_row0(embeddings_table)
    out_sc = _make_sc_gather(B, Bsc, num_units)(table, idx[:Bsc])
    out = _make_tc_fill(B, Bsc, num_units)(idx.reshape(-1, 1), table, out_sc)
    return out.reshape(bsz, seq, seq2, num_units)


# speedup vs baseline: 2.6409x; 1.1703x over previous
import functools

import jax
import jax.numpy as jnp
from jax import lax
from jax.experimental import pallas as pl
from jax.experimental.pallas import tpu as pltpu
from jax.experimental.pallas import tpu_sc as plsc

NUM_UNITS = 768
NUM_REL = 129


def _zero_row0_body(table_ref, out_ref):
    rows = lax.broadcasted_iota(jnp.int32, table_ref.shape, 0)
    out_ref[...] = jnp.where(rows == 0, jnp.float32(0.0), table_ref[...])


def _zero_row0(table):
    return pl.pallas_call(
        _zero_row0_body,
        out_shape=jax.ShapeDtypeStruct(table.shape, table.dtype),
    )(table)


def _onehot_body(idx_ref, table_ref, out_ref):
    idx = idx_ref[...]
    classes = lax.broadcasted_iota(jnp.int32, (1, NUM_REL), 1)
    oh = jnp.where((idx == classes) & (idx >= 1), jnp.float32(1.0),
                   jnp.float32(0.0))
    out_ref[...] = jnp.dot(oh, table_ref[...],
                           preferred_element_type=jnp.float32)


def _onehot_fill_body(idx_ref, table_ref, _prev_ref, out_ref):
    _onehot_body(idx_ref, table_ref, out_ref)


@functools.lru_cache(maxsize=None)
def _make_tc_fill(B, Bsc, D):
    RB = 4096
    nblk0 = Bsc // RB
    grid = ((B - Bsc) // RB,)
    return pl.pallas_call(
        _onehot_fill_body,
        grid=grid,
        in_specs=[
            pl.BlockSpec((RB, 1), lambda i: (i + nblk0, 0)),
            pl.BlockSpec((NUM_REL, D), lambda i: (0, 0)),
            pl.BlockSpec((8, 128), lambda i: (0, 0)),
        ],
        out_specs=pl.BlockSpec((RB, D), lambda i: (i + nblk0, 0)),
        out_shape=jax.ShapeDtypeStruct((B, D), jnp.float32),
        input_output_aliases={2: 0},
    )


@functools.lru_cache(maxsize=None)
def _make_sc_gather(B, Bsc, D):
    info = plsc.get_sparse_core_info()
    NC, NS = info.num_cores, info.num_subcores
    NW = NC * NS
    b_per_w = Bsc // NW
    C = 32
    NBUF = 4
    AHEAD = 2
    nch = b_per_w // C
    assert b_per_w % C == 0 and nch % NBUF == 0
    assert Bsc % NW == 0

    mesh = plsc.VectorSubcoreMesh(core_axis_name="c", subcore_axis_name="s")

    @functools.partial(
        pl.kernel,
        mesh=mesh,
        out_type=jax.ShapeDtypeStruct((B, D), jnp.float32),
        scratch_types=(
            [pltpu.VMEM((b_per_w,), jnp.int32)]
            + [pltpu.VMEM((C, D), jnp.float32)] * NBUF
            + [pltpu.SemaphoreType.DMA] * (2 * NBUF)
        ),
    )
    def gather_kernel(table_hbm, idx_hbm, out_hbm, idx_v, *bufs_and_sems):
        rows = bufs_and_sems[:NBUF]
        gsem = bufs_and_sems[NBUF:2 * NBUF]
        wsem = bufs_and_sems[2 * NBUF:]
        wid = lax.axis_index("s") * NC + lax.axis_index("c")
        base = wid * b_per_w
        pltpu.sync_copy(idx_hbm.at[pl.ds(base, b_per_w)], idx_v)

        def g_copy(c, b):
            return pltpu.make_async_copy(
                table_hbm.at[idx_v.at[pl.ds(c * C, C)]], rows[b], gsem[b])

        def w_copy(c, b):
            return pltpu.make_async_copy(
                rows[b], out_hbm.at[pl.ds(base + c * C, C)], wsem[b])

        for k in range(AHEAD):
            g_copy(k, k).start()

        def loop_body(g):
            for b in range(NBUF):
                c = g + b
                g_copy(c, b).wait()
                w_copy(c, b).start()

                @pl.when(c >= AHEAD)
                def _():
                    w_copy(c - AHEAD, (b - AHEAD) % NBUF).wait()

                @pl.when(c + AHEAD < nch)
                def _():
                    g_copy(c + AHEAD, (b + AHEAD) % NBUF).start()

        pl.loop(0, nch, step=NBUF)(loop_body)
        for k in range(AHEAD):
            c = nch - AHEAD + k
            w_copy(c, c % NBUF).wait()

    return gather_kernel


SC_FRAC_NUM = 1
SC_FRAC_DEN = 8


def kernel(relation_matrix, embeddings_table):
    bsz, seq, seq2 = relation_matrix.shape
    num_units = embeddings_table.shape[1]
    B = bsz * seq * seq2
    Bsc = (B * SC_FRAC_NUM // SC_FRAC_DEN) // 8192 * 8192
    idx = relation_matrix.reshape(-1)
    table = _zero_row0(embeddings_table)
    out_sc = _make_sc_gather(B, Bsc, num_units)(table, idx[:Bsc])
    out = _make_tc_fill(B, Bsc, num_units)(idx.reshape(-1, 1), table, out_sc)
    return out.reshape(bsz, seq, seq2, num_units)
